# baseline (speedup 1.0000x reference)
import functools

import jax
import jax.numpy as jnp
from jax import lax
from jax.experimental import pallas as pl
from jax.experimental.pallas import tpu as pltpu
from jax.experimental.pallas import tpu_sc as plsc

D_MODEL = 64
SCALE = 8.0

NUM_CORES = 2
NUM_SUBCORES = 16
NUM_WORKERS = NUM_CORES * NUM_SUBCORES

BBLK = 128
PITCH = BBLK + 1
NBUF = 4


@functools.lru_cache(maxsize=None)
def _build(S, B):
    assert B == NUM_WORKERS * BBLK
    assert S % NBUF == 0

    mesh = plsc.VectorSubcoreMesh(
        core_axis_name="c",
        subcore_axis_name="s",
        num_cores=NUM_CORES,
        num_subcores=NUM_SUBCORES,
    )

    @functools.partial(
        pl.kernel,
        out_type=jax.ShapeDtypeStruct((S, 8, NUM_WORKERS, 8, BBLK), jnp.float32),
        mesh=mesh,
        scratch_types=[
            pltpu.VMEM((S, BBLK), jnp.int32),
            [pltpu.VMEM((BBLK, D_MODEL), jnp.float32) for _ in range(NBUF)],
            [pltpu.VMEM((8, 8, PITCH), jnp.float32) for _ in range(NBUF)],
            [pltpu.SemaphoreType.DMA for _ in range(NBUF)],
            [pltpu.SemaphoreType.DMA for _ in range(NBUF)],
        ],
        compiler_params=pltpu.CompilerParams(
            use_tc_tiling_on_sc=False, needs_layout_passes=False
        ),
    )
    def emb_kernel(xt_hbm, table_hbm, out_hbm, idx_v, gbufs, obufs, gsems, osems):
        w = lax.axis_index("s") * NUM_CORES + lax.axis_index("c")
        pltpu.sync_copy(xt_hbm.at[:, pl.ds(w * BBLK, BBLK)], idx_v)

        def gather(s, b):
            return pltpu.make_async_copy(
                table_hbm.at[idx_v.at[s]], gbufs[b], gsems[b]
            )

        def out_write(s, b):
            return pltpu.make_async_copy(
                obufs[b].at[:, :, pl.ds(0, BBLK)],
                out_hbm.at[s, :, w],
                osems[b],
            )

        for b in range(NBUF):
            gather(b, b).start()

        iota = lax.iota(jnp.int32, 16)
        tcs = [(iota + 16 * m) // 8 for m in range(D_MODEL // 16)]
        rs = [(iota + 16 * m) % 8 for m in range(D_MODEL // 16)]

        @pl.loop(0, S, step=NBUF)
        def _group(g):
            for b in range(NBUF):
                s = g + b
                gather(s, b).wait()

                @pl.when(s >= NBUF)
                def _():
                    out_write(s - NBUF, b).wait()

                @pl.loop(0, BBLK, unroll=2)
                def _row(l):
                    lv = jnp.full((16,), 0, jnp.int32) + l
                    for m in range(D_MODEL // 16):
                        vals = gbufs[b][l, pl.ds(16 * m, 16)] * SCALE
                        plsc.store_scatter(obufs[b], [tcs[m], rs[m], lv], vals)

                @pl.when(s + NBUF < S)
                def _():
                    gather(s + NBUF, b).start()

                out_write(s, b).start()

        for b in range(NBUF):
            out_write(S - NBUF + b, b).wait()

    return emb_kernel


def kernel(x, lut):
    bsz, seq = x.shape
    xt = jnp.transpose(x)
    out5 = _build(seq, bsz)(xt, lut)
    out = out5.transpose(2, 4, 0, 1, 3).reshape(bsz, seq, D_MODEL)
    return out


# speedup vs baseline: 1.0587x; 1.0587x over previous
import functools

import jax
import jax.numpy as jnp
from jax import lax
from jax.experimental import pallas as pl
from jax.experimental.pallas import tpu as pltpu
from jax.experimental.pallas import tpu_sc as plsc

D_MODEL = 64
SCALE = 8.0

NUM_CORES = 2
NUM_SUBCORES = 16
NUM_WORKERS = NUM_CORES * NUM_SUBCORES

BBLK = 128
PITCH = BBLK + 1
NBUF = 4


@functools.lru_cache(maxsize=None)
def _build(S, B):
    assert B == NUM_WORKERS * BBLK
    assert S % NBUF == 0

    mesh = plsc.VectorSubcoreMesh(
        core_axis_name="c",
        subcore_axis_name="s",
        num_cores=NUM_CORES,
        num_subcores=NUM_SUBCORES,
    )

    @functools.partial(
        pl.kernel,
        out_type=jax.ShapeDtypeStruct((S, 8, NUM_WORKERS, 8, BBLK), jnp.float32),
        mesh=mesh,
        scratch_types=[
            pltpu.VMEM((S, BBLK), jnp.int32),
            [pltpu.VMEM((BBLK, 2 * D_MODEL), jnp.float32) for _ in range(NBUF)],
            [pltpu.VMEM((8, 8, PITCH), jnp.float32) for _ in range(NBUF)],
            [pltpu.SemaphoreType.DMA for _ in range(NBUF)],
            [pltpu.SemaphoreType.DMA for _ in range(NBUF)],
        ],
        compiler_params=pltpu.CompilerParams(
            use_tc_tiling_on_sc=False, needs_layout_passes=False
        ),
    )
    def emb_kernel(xt_hbm, table_hbm, out_hbm, idx_v, gbufs, obufs, gsems, osems):
        w = lax.axis_index("s") * NUM_CORES + lax.axis_index("c")
        pltpu.sync_copy(xt_hbm.at[:, pl.ds(w * BBLK, BBLK)], idx_v)

        def gather(s, b):
            return pltpu.make_async_copy(
                table_hbm.at[idx_v.at[s]], gbufs[b], gsems[b]
            )

        def out_write(s, b):
            return pltpu.make_async_copy(
                obufs[b].at[:, :, pl.ds(0, BBLK)],
                out_hbm.at[s, :, w],
                osems[b],
            )

        for b in range(NBUF):
            gather(b, b).start()

        iota = lax.iota(jnp.int32, 16)
        tcs = [(iota + 16 * m) // 8 for m in range(D_MODEL // 16)]
        rs = [(iota + 16 * m) % 8 for m in range(D_MODEL // 16)]

        @pl.loop(0, S, step=NBUF)
        def _group(g):
            for b in range(NBUF):
                s = g + b
                gather(s, b).wait()

                @pl.when(s >= NBUF)
                def _():
                    out_write(s - NBUF, b).wait()

                @pl.loop(0, BBLK, unroll=2)
                def _row(l):
                    lv = jnp.full((16,), 0, jnp.int32) + l
                    for m in range(D_MODEL // 16):
                        vals = gbufs[b][l, pl.ds(16 * m, 16)] * SCALE
                        plsc.store_scatter(obufs[b], [tcs[m], rs[m], lv], vals)

                @pl.when(s + NBUF < S)
                def _():
                    gather(s + NBUF, b).start()

                out_write(s, b).start()

        for b in range(NBUF):
            out_write(S - NBUF + b, b).wait()

    return emb_kernel


def kernel(x, lut):
    bsz, seq = x.shape
    xt = jnp.transpose(x)
    lutp = jnp.pad(lut, ((0, 0), (0, D_MODEL)))
    out5 = _build(seq, bsz)(xt, lutp)
    out = out5.transpose(2, 4, 0, 1, 3).reshape(bsz, seq, D_MODEL)
    return out


# speedup vs baseline: 1.3110x; 1.2383x over previous
import functools

import jax
import jax.numpy as jnp
from jax import lax
from jax.experimental import pallas as pl
from jax.experimental.pallas import tpu as pltpu
from jax.experimental.pallas import tpu_sc as plsc

D_MODEL = 64
SCALE = 8.0

NUM_CORES = 2
NUM_SUBCORES = 16
NUM_WORKERS = NUM_CORES * NUM_SUBCORES

BBLK = 128
PITCH = BBLK + 1
NBUF = 4


@functools.lru_cache(maxsize=None)
def _build(S, B):
    assert B == NUM_WORKERS * BBLK
    assert S % NBUF == 0

    mesh = plsc.VectorSubcoreMesh(
        core_axis_name="c",
        subcore_axis_name="s",
        num_cores=NUM_CORES,
        num_subcores=NUM_SUBCORES,
    )

    @functools.partial(
        pl.kernel,
        out_type=jax.ShapeDtypeStruct((S, 8, NUM_WORKERS, 8, BBLK), jnp.float32),
        mesh=mesh,
        scratch_types=[
            pltpu.VMEM((S, BBLK), jnp.int32),
            [pltpu.VMEM((BBLK, 2 * D_MODEL), jnp.float32) for _ in range(NBUF)],
            [pltpu.VMEM((8, 8, PITCH), jnp.float32) for _ in range(NBUF)],
            [pltpu.SemaphoreType.DMA for _ in range(NBUF)],
            [pltpu.SemaphoreType.DMA for _ in range(NBUF)],
        ],
        compiler_params=pltpu.CompilerParams(
            use_tc_tiling_on_sc=False, needs_layout_passes=False
        ),
    )
    def emb_kernel(xt_hbm, table_hbm, out_hbm, idx_v, gbufs, obufs, gsems, osems):
        w = lax.axis_index("s") * NUM_CORES + lax.axis_index("c")
        pltpu.sync_copy(xt_hbm.at[:, pl.ds(w * BBLK, BBLK)], idx_v)

        def gather(s, b):
            return pltpu.make_async_copy(
                table_hbm.at[idx_v.at[s]], gbufs[b], gsems[b]
            )

        def out_write(s, b):
            return pltpu.make_async_copy(
                obufs[b].at[:, :, pl.ds(0, BBLK)],
                out_hbm.at[s, :, w],
                osems[b],
            )

        for b in range(NBUF):
            gather(b, b).start()

        iota = lax.iota(jnp.int32, 16)
        tcs = [(iota + 16 * m) // 8 for m in range(D_MODEL // 16)]
        rs = [(iota + 16 * m) % 8 for m in range(D_MODEL // 16)]

        @pl.loop(0, S, step=NBUF)
        def _group(g):
            for b in range(NBUF):
                s = g + b
                gather(s, b).wait()

                @pl.when(s >= NBUF)
                def _():
                    out_write(s - NBUF, b).wait()

                @pl.loop(0, BBLK, unroll=4)
                def _row(l):
                    lv = jnp.full((16,), 0, jnp.int32) + l
                    nm = D_MODEL // 16
                    vals = [gbufs[b][l, pl.ds(16 * m, 16)] for m in range(nm)]
                    scaled = [v * SCALE for v in vals]
                    for m in range(nm):
                        plsc.store_scatter(
                            obufs[b], [tcs[m], rs[m], lv], scaled[m]
                        )

                @pl.when(s + NBUF < S)
                def _():
                    gather(s + NBUF, b).start()

                out_write(s, b).start()

        for b in range(NBUF):
            out_write(S - NBUF + b, b).wait()

    return emb_kernel


def kernel(x, lut):
    bsz, seq = x.shape
    xt = jnp.transpose(x)
    lutp = jnp.pad(lut, ((0, 0), (0, D_MODEL)))
    out5 = _build(seq, bsz)(xt, lutp)
    out = out5.transpose(2, 4, 0, 1, 3).reshape(bsz, seq, D_MODEL)
    return out


# speedup vs baseline: 1.3811x; 1.0535x over previous
import functools

import jax
import jax.numpy as jnp
from jax import lax
from jax.experimental import pallas as pl
from jax.experimental.pallas import tpu as pltpu
from jax.experimental.pallas import tpu_sc as plsc

D_MODEL = 64
SCALE = 8.0

NUM_CORES = 2
NUM_SUBCORES = 16
NUM_WORKERS = NUM_CORES * NUM_SUBCORES

BBLK = 128
PITCH = BBLK + 1
NBUF = 4


@functools.lru_cache(maxsize=None)
def _build(S, B):
    assert B == NUM_WORKERS * BBLK
    assert S % NBUF == 0

    mesh = plsc.VectorSubcoreMesh(
        core_axis_name="c",
        subcore_axis_name="s",
        num_cores=NUM_CORES,
        num_subcores=NUM_SUBCORES,
    )

    @functools.partial(
        pl.kernel,
        out_type=jax.ShapeDtypeStruct((S, 8, NUM_WORKERS, 8, BBLK), jnp.float32),
        mesh=mesh,
        scratch_types=[
            pltpu.VMEM((S, BBLK), jnp.int32),
            [pltpu.VMEM((BBLK, 2 * D_MODEL), jnp.float32) for _ in range(NBUF)],
            [pltpu.VMEM((8, 8, PITCH), jnp.float32) for _ in range(NBUF)],
            [pltpu.SemaphoreType.DMA for _ in range(NBUF)],
            [pltpu.SemaphoreType.DMA for _ in range(NBUF)],
        ],
        compiler_params=pltpu.CompilerParams(
            use_tc_tiling_on_sc=False, needs_layout_passes=False
        ),
    )
    def emb_kernel(xt_hbm, table_hbm, out_hbm, idx_v, gbufs, obufs, gsems, osems):
        w = lax.axis_index("s") * NUM_CORES + lax.axis_index("c")
        pltpu.sync_copy(xt_hbm.at[:, pl.ds(w * BBLK, BBLK)], idx_v)

        def gather(s, b):
            return pltpu.make_async_copy(
                table_hbm.at[idx_v.at[s]], gbufs[b], gsems[b]
            )

        def out_write(s, b):
            return pltpu.make_async_copy(
                obufs[b].at[:, :, pl.ds(0, BBLK)],
                out_hbm.at[s, :, w],
                osems[b],
            )

        for b in range(NBUF):
            gather(b, b).start()

        iota = lax.iota(jnp.int32, 16)
        tcs = [(iota + 16 * m) // 8 for m in range(D_MODEL // 16)]
        rs = [(iota + 16 * m) % 8 for m in range(D_MODEL // 16)]

        @pl.loop(0, S, step=NBUF)
        def _group(g):
            for b in range(NBUF):
                s = g + b
                gather(s, b).wait()

                @pl.when(s >= NBUF)
                def _():
                    out_write(s - NBUF, b).wait()

                @pl.loop(0, BBLK, step=2, unroll=2)
                def _row(l):
                    nm = D_MODEL // 16
                    lv = jnp.full((16,), 0, jnp.int32) + l
                    lv2 = lv + 1
                    vals = [gbufs[b][l, pl.ds(16 * m, 16)] for m in range(nm)]
                    vals2 = [
                        gbufs[b][l + 1, pl.ds(16 * m, 16)] for m in range(nm)
                    ]
                    sc1 = [v * SCALE for v in vals]
                    sc2 = [v * SCALE for v in vals2]
                    for m in range(nm):
                        plsc.store_scatter(obufs[b], [tcs[m], rs[m], lv], sc1[m])
                    for m in range(nm):
                        plsc.store_scatter(obufs[b], [tcs[m], rs[m], lv2], sc2[m])

                @pl.when(s + NBUF < S)
                def _():
                    gather(s + NBUF, b).start()

                out_write(s, b).start()

        for b in range(NBUF):
            out_write(S - NBUF + b, b).wait()

    return emb_kernel


def kernel(x, lut):
    bsz, seq = x.shape
    xt = jnp.transpose(x)
    lutp = jnp.pad(lut, ((0, 0), (0, D_MODEL)))
    out5 = _build(seq, bsz)(xt, lutp)
    out = out5.transpose(2, 4, 0, 1, 3).reshape(bsz, seq, D_MODEL)
    return out


# speedup vs baseline: 1.3821x; 1.0007x over previous
import functools

import jax
import jax.numpy as jnp
from jax import lax
from jax.experimental import pallas as pl
from jax.experimental.pallas import tpu as pltpu
from jax.experimental.pallas import tpu_sc as plsc

D_MODEL = 64
SCALE = 8.0

NUM_CORES = 2
NUM_SUBCORES = 16
NUM_WORKERS = NUM_CORES * NUM_SUBCORES

BBLK = 128
PITCH = BBLK + 1
NBUF = 4


@functools.lru_cache(maxsize=None)
def _build(S, B):
    assert B == NUM_WORKERS * BBLK
    assert S % NBUF == 0

    mesh = plsc.VectorSubcoreMesh(
        core_axis_name="c",
        subcore_axis_name="s",
        num_cores=NUM_CORES,
        num_subcores=NUM_SUBCORES,
    )

    @functools.partial(
        pl.kernel,
        out_type=jax.ShapeDtypeStruct((S, 8, NUM_WORKERS, 8, BBLK), jnp.float32),
        mesh=mesh,
        scratch_types=[
            pltpu.VMEM((S, BBLK), jnp.int32),
            [pltpu.VMEM((BBLK, 2 * D_MODEL), jnp.float32) for _ in range(NBUF)],
            [pltpu.VMEM((8, 8, PITCH), jnp.float32) for _ in range(NBUF)],
            [pltpu.SemaphoreType.DMA for _ in range(NBUF)],
            [pltpu.SemaphoreType.DMA for _ in range(NBUF)],
        ],
        compiler_params=pltpu.CompilerParams(
            use_tc_tiling_on_sc=False, needs_layout_passes=False
        ),
    )
    def emb_kernel(xt_hbm, table_hbm, out_hbm, idx_v, gbufs, obufs, gsems, osems):
        w = lax.axis_index("s") * NUM_CORES + lax.axis_index("c")
        pltpu.sync_copy(xt_hbm.at[:, pl.ds(w * BBLK, BBLK)], idx_v)

        def gather(s, b):
            return pltpu.make_async_copy(
                table_hbm.at[idx_v.at[s]], gbufs[b], gsems[b]
            )

        def out_write(s, b):
            return pltpu.make_async_copy(
                obufs[b].at[:, :, pl.ds(0, BBLK)],
                out_hbm.at[s, :, w],
                osems[b],
            )

        for b in range(NBUF):
            gather(b, b).start()

        iota = lax.iota(jnp.int32, 16)
        tcs = [(iota + 16 * m) // 8 for m in range(D_MODEL // 16)]
        rs = [(iota + 16 * m) % 8 for m in range(D_MODEL // 16)]

        @pl.loop(0, S, step=NBUF)
        def _group(g):
            for b in range(NBUF):
                s = g + b
                gather(s, b).wait()

                @pl.when(s >= NBUF)
                def _():
                    out_write(s - NBUF, b).wait()

                @pl.loop(0, BBLK, step=4, unroll=1)
                def _row(l):
                    nm = D_MODEL // 16
                    lv0 = jnp.full((16,), 0, jnp.int32) + l
                    lvs = [lv0, lv0 + 1, lv0 + 2, lv0 + 3]
                    vals = [
                        [gbufs[b][l + r, pl.ds(16 * m, 16)] for m in range(nm)]
                        for r in range(4)
                    ]
                    sc = [[v * SCALE for v in row] for row in vals]
                    for r in range(4):
                        for m in range(nm):
                            plsc.store_scatter(
                                obufs[b], [tcs[m], rs[m], lvs[r]], sc[r][m]
                            )

                @pl.when(s + NBUF < S)
                def _():
                    gather(s + NBUF, b).start()

                out_write(s, b).start()

        for b in range(NBUF):
            out_write(S - NBUF + b, b).wait()

    return emb_kernel


def kernel(x, lut):
    bsz, seq = x.shape
    xt = jnp.transpose(x)
    lutp = jnp.pad(lut, ((0, 0), (0, D_MODEL)))
    out5 = _build(seq, bsz)(xt, lutp)
    out = out5.transpose(2, 4, 0, 1, 3).reshape(bsz, seq, D_MODEL)
    return out


# speedup vs baseline: 1.3824x; 1.0002x over previous
import functools

import jax
import jax.numpy as jnp
from jax import lax
from jax.experimental import pallas as pl
from jax.experimental.pallas import tpu as pltpu
from jax.experimental.pallas import tpu_sc as plsc

D_MODEL = 64
SCALE = 8.0

NUM_CORES = 2
NUM_SUBCORES = 16
NUM_WORKERS = NUM_CORES * NUM_SUBCORES

BBLK = 128
PITCH = BBLK + 1
NBUF = 4


@functools.lru_cache(maxsize=None)
def _build(S, B):
    assert B == NUM_WORKERS * BBLK
    assert S % NBUF == 0

    mesh = plsc.VectorSubcoreMesh(
        core_axis_name="c",
        subcore_axis_name="s",
        num_cores=NUM_CORES,
        num_subcores=NUM_SUBCORES,
    )

    @functools.partial(
        pl.kernel,
        out_type=jax.ShapeDtypeStruct((S, 8, NUM_WORKERS, 8, BBLK), jnp.float32),
        mesh=mesh,
        scratch_types=[
            pltpu.VMEM((S, BBLK), jnp.int32),
            [pltpu.VMEM((BBLK, 2 * D_MODEL), jnp.float32) for _ in range(NBUF)],
            [pltpu.VMEM((8, 8, PITCH), jnp.float32) for _ in range(NBUF)],
            [pltpu.SemaphoreType.DMA for _ in range(NBUF)],
            [pltpu.SemaphoreType.DMA for _ in range(NBUF)],
        ],
        compiler_params=pltpu.CompilerParams(
            use_tc_tiling_on_sc=False, needs_layout_passes=False
        ),
    )
    def emb_kernel(xt_hbm, table_hbm, out_hbm, idx_v, gbufs, obufs, gsems, osems):
        w = lax.axis_index("s") * NUM_CORES + lax.axis_index("c")
        pltpu.sync_copy(xt_hbm.at[:, pl.ds(w * BBLK, BBLK)], idx_v)

        def gather(s, b):
            return pltpu.make_async_copy(
                table_hbm.at[idx_v.at[s]], gbufs[b], gsems[b]
            )

        def out_write(s, b):
            return pltpu.make_async_copy(
                obufs[b].at[:, :, pl.ds(0, BBLK)],
                out_hbm.at[s, :, w],
                osems[b],
            )

        for b in range(NBUF):
            gather(b, b).start()

        iota = lax.iota(jnp.int32, 16)
        tcs = [(iota + 16 * m) // 8 for m in range(D_MODEL // 16)]
        rs = [(iota + 16 * m) % 8 for m in range(D_MODEL // 16)]

        @pl.loop(0, S, step=NBUF)
        def _group(g):
            for b in range(NBUF):
                s = g + b
                gather(s, b).wait()

                @pl.when(s >= NBUF)
                def _():
                    out_write(s - NBUF, b).wait()

                @pl.loop(0, BBLK, step=2, unroll=2)
                def _row(l):
                    nm = D_MODEL // 16
                    lv = jnp.full((16,), 0, jnp.int32) + l
                    lv2 = lv + 1
                    vals = [gbufs[b][l, pl.ds(16 * m, 16)] for m in range(nm)]
                    vals2 = [
                        gbufs[b][l + 1, pl.ds(16 * m, 16)] for m in range(nm)
                    ]
                    sc1 = [v * SCALE for v in vals]
                    sc2 = [v * SCALE for v in vals2]
                    for m in range(nm):
                        plsc.store_scatter(obufs[b], [tcs[m], rs[m], lv], sc1[m])
                    for m in range(nm):
                        plsc.store_scatter(obufs[b], [tcs[m], rs[m], lv2], sc2[m])

                @pl.when(s + NBUF < S)
                def _():
                    gather(s + NBUF, b).start()

                out_write(s, b).start()

        for b in range(NBUF):
            out_write(S - NBUF + b, b).wait()

    return emb_kernel


def kernel(x, lut):
    bsz, seq = x.shape
    xt = jnp.transpose(x)
    lutp = jnp.pad(lut, ((0, 0), (0, D_MODEL)))
    out5 = _build(seq, bsz)(xt, lutp)
    out = out5.transpose(2, 4, 0, 1, 3).reshape(bsz, seq, D_MODEL)
    return out
